# Initial kernel scaffold; baseline (speedup 1.0000x reference)
import jax
import jax.numpy as jnp
from jax.experimental import pallas as pl


def kernel(key_soft, u_t, keys_w, r_keys_w, vparams_w):
    raise NotImplementedError("write your pallas kernel here")


# speedup vs baseline: 1.3900x; 1.3900x over previous
import functools

import jax
import jax.numpy as jnp
from jax import lax
from jax.experimental import pallas as pl

B, T = 16, 576
KEY_DIM = 256
N_E = 1024
E_DIM = 256
E_SPLIT = 4
KT = 0.1
EPS = 1e-12


def _fused_body(ks_ref, keys_ref, r_ref, vp_ref,
                vs_ref, vh_ref, idx_ref, score_ref):
    x = ks_ref[0]
    xn = jnp.sqrt(jnp.sum(x * x, axis=1, keepdims=True))
    x = x / jnp.maximum(xn, EPS)

    k = keys_ref[...]
    kn = jnp.sqrt(jnp.sum(k * k, axis=1, keepdims=True))
    r = jnp.clip(r_ref[...], 0.0, 1.0)
    ks = k * (r / jnp.maximum(kn, EPS))

    score = lax.dot_general(x, ks, (((1,), (1,)), ((), ())),
                            preferred_element_type=jnp.float32)
    score_ref[0] = score

    m = jnp.max(score, axis=1, keepdims=True)
    iota = lax.broadcasted_iota(jnp.int32, (T, N_E), 1)
    hit = score == m
    idx = jnp.min(jnp.where(hit, iota, N_E), axis=1)
    idx_ref[0, 0] = idx

    w = jnp.exp((score - m) * (1.0 / KT))
    w = w / jnp.sum(w, axis=1, keepdims=True)

    v = vp_ref[...]
    v4 = v.reshape(N_E, E_SPLIT, E_DIM // E_SPLIT)
    vn = jnp.sqrt(jnp.sum(v4 * v4, axis=2, keepdims=True))
    vpn = (v4 / jnp.maximum(vn, EPS)).reshape(N_E, E_DIM)

    vs_ref[0] = lax.dot_general(w, vpn, (((1,), (0,)), ((), ())),
                                preferred_element_type=jnp.float32)

    onehot = (iota == idx[:, None]).astype(jnp.float32)
    vh_ref[0] = lax.dot_general(onehot, vpn, (((1,), (0,)), ((), ())),
                                preferred_element_type=jnp.float32)


@functools.partial(jax.jit, static_argnames=("interpret",))
def _run(key_soft, keys_w, r_keys_w, vparams_w, interpret=False):
    grid = (B,)
    out_shapes = (
        jax.ShapeDtypeStruct((B, T, E_DIM), jnp.float32),
        jax.ShapeDtypeStruct((B, T, E_DIM), jnp.float32),
        jax.ShapeDtypeStruct((B, 1, T), jnp.int32),
        jax.ShapeDtypeStruct((B, T, N_E), jnp.float32),
    )
    in_specs = [
        pl.BlockSpec((1, T, KEY_DIM), lambda i: (i, 0, 0)),
        pl.BlockSpec((N_E, KEY_DIM), lambda i: (0, 0)),
        pl.BlockSpec((N_E, 1), lambda i: (0, 0)),
        pl.BlockSpec((N_E, E_DIM), lambda i: (0, 0)),
    ]
    out_specs = (
        pl.BlockSpec((1, T, E_DIM), lambda i: (i, 0, 0)),
        pl.BlockSpec((1, T, E_DIM), lambda i: (i, 0, 0)),
        pl.BlockSpec((1, 1, T), lambda i: (i, 0, 0)),
        pl.BlockSpec((1, T, N_E), lambda i: (i, 0, 0)),
    )
    return pl.pallas_call(
        _fused_body,
        grid=grid,
        in_specs=in_specs,
        out_specs=out_specs,
        out_shape=out_shapes,
        interpret=interpret,
    )(key_soft, keys_w, r_keys_w, vparams_w)


def kernel(key_soft, u_t, keys_w, r_keys_w, vparams_w):
    v_soft, v_hard, idx, score = _run(key_soft, keys_w, r_keys_w, vparams_w)
    return v_soft, v_hard, idx.reshape(B, T), score


# speedup vs baseline: 2.8699x; 2.0648x over previous
import functools

import jax
import jax.numpy as jnp
from jax import lax
from jax.experimental import pallas as pl

B, T = 16, 576
KEY_DIM = 256
N_E = 1024
E_DIM = 256
E_SPLIT = 4
KT = 0.1
EPS = 1e-12


def _prep_body(keys_ref, r_ref, vp_ref, ks_ref, vpn_ref):
    k = keys_ref[...]
    kn = jnp.sqrt(jnp.sum(k * k, axis=1, keepdims=True))
    r = jnp.clip(r_ref[...], 0.0, 1.0)
    ks_ref[...] = k * (r / jnp.maximum(kn, EPS))
    v = vp_ref[...]
    v4 = v.reshape(N_E, E_SPLIT, E_DIM // E_SPLIT)
    vn = jnp.sqrt(jnp.sum(v4 * v4, axis=2, keepdims=True))
    vpn_ref[...] = (v4 / jnp.maximum(vn, EPS)).reshape(N_E, E_DIM)


def _fused_body(x_ref, ks_ref, vpn_ref, vs_ref, vh_ref, idx_ref, score_ref):
    x = x_ref[0]
    xn = jnp.sqrt(jnp.sum(x * x, axis=1, keepdims=True))
    x = x / jnp.maximum(xn, EPS)

    score = lax.dot_general(x, ks_ref[...], (((1,), (1,)), ((), ())),
                            preferred_element_type=jnp.float32)
    score_ref[0] = score

    m = jnp.max(score, axis=1, keepdims=True)
    iota = lax.broadcasted_iota(jnp.int32, (T, N_E), 1)
    idx = jnp.min(jnp.where(score == m, iota, N_E), axis=1)
    idx_ref[0, 0] = idx

    e = jnp.exp((score - m) * (1.0 / KT))
    s = jnp.sum(e, axis=1, keepdims=True)

    vpn = vpn_ref[...]
    acc = lax.dot_general(e, vpn, (((1,), (0,)), ((), ())),
                          preferred_element_type=jnp.float32)
    vs_ref[0] = acc / s

    onehot = (iota == idx[:, None]).astype(jnp.float32)
    vh_ref[0] = lax.dot_general(onehot, vpn, (((1,), (0,)), ((), ())),
                                preferred_element_type=jnp.float32)


@functools.partial(jax.jit, static_argnames=("interpret",))
def _run(key_soft, keys_w, r_keys_w, vparams_w, interpret=False):
    ks_scaled, vpn = pl.pallas_call(
        _prep_body,
        out_shape=(
            jax.ShapeDtypeStruct((N_E, KEY_DIM), jnp.float32),
            jax.ShapeDtypeStruct((N_E, E_DIM), jnp.float32),
        ),
        interpret=interpret,
    )(keys_w, r_keys_w, vparams_w)

    out_shapes = (
        jax.ShapeDtypeStruct((B, T, E_DIM), jnp.float32),
        jax.ShapeDtypeStruct((B, T, E_DIM), jnp.float32),
        jax.ShapeDtypeStruct((B, 1, T), jnp.int32),
        jax.ShapeDtypeStruct((B, T, N_E), jnp.float32),
    )
    in_specs = [
        pl.BlockSpec((1, T, KEY_DIM), lambda i: (i, 0, 0)),
        pl.BlockSpec((N_E, KEY_DIM), lambda i: (0, 0)),
        pl.BlockSpec((N_E, E_DIM), lambda i: (0, 0)),
    ]
    out_specs = (
        pl.BlockSpec((1, T, E_DIM), lambda i: (i, 0, 0)),
        pl.BlockSpec((1, T, E_DIM), lambda i: (i, 0, 0)),
        pl.BlockSpec((1, 1, T), lambda i: (i, 0, 0)),
        pl.BlockSpec((1, T, N_E), lambda i: (i, 0, 0)),
    )
    return pl.pallas_call(
        _fused_body,
        grid=(B,),
        in_specs=in_specs,
        out_specs=out_specs,
        out_shape=out_shapes,
        interpret=interpret,
    )(key_soft, ks_scaled, vpn)


def kernel(key_soft, u_t, keys_w, r_keys_w, vparams_w):
    v_soft, v_hard, idx, score = _run(key_soft, keys_w, r_keys_w, vparams_w)
    return v_soft, v_hard, idx.reshape(B, T), score
